# baseline (speedup 1.0000x reference)
import jax
import jax.numpy as jnp
from jax import lax
from jax.experimental import pallas as pl
from jax.experimental.pallas import tpu as pltpu

_TILE_ROWS = 8192
_VMEM_LIMIT_BYTES = 56 * 1024 * 1024


def _cdiv(a, b):
    return -(-a // b)


def _pad_rows(a, rows_padded):
    r = a.shape[0]
    if rows_padded != r:
        a = jnp.pad(a, ((0, rows_padded - r), (0, 0)))
    return a


def _make_phased_kernel(tiles, inv_cnt, eps):
    def _phased_kernel(x_ref, wn_ref, bn_ref, e_ref, we_ref, g_ref, b_ref,
                       xo_ref, s1_ref, s2_ref, eo_ref):
        j = pl.program_id(0)

        @pl.when(j == 0)
        def _init():
            s1_ref[...] = jnp.zeros_like(s1_ref)
            s2_ref[...] = jnp.zeros_like(s2_ref)

        @pl.when(j < tiles)
        def _phase_a():
            xo_ref[...] = (jnp.dot(x_ref[...], wn_ref[...],
                                   preferred_element_type=jnp.float32)
                           + bn_ref[...]).astype(xo_ref.dtype)
            acc = jnp.dot(e_ref[...], we_ref[...],
                          preferred_element_type=jnp.float32)
            s1_ref[...] += jnp.sum(acc, axis=0, keepdims=True)
            s2_ref[...] += jnp.sum(acc * acc, axis=0, keepdims=True)

        @pl.when(j >= tiles)
        def _phase_b():
            mu = s1_ref[...] * inv_cnt
            var = jnp.maximum(s2_ref[...] * inv_cnt - mu * mu, 0.0)
            scale = g_ref[...] * lax.rsqrt(var + eps)
            shift = b_ref[...] - mu * scale
            acc = jnp.dot(e_ref[...], we_ref[...],
                          preferred_element_type=jnp.float32)
            eo_ref[...] = (acc * scale + shift).astype(eo_ref.dtype)
    return _phased_kernel


def kernel(x, edge_attr, w_node, b_node, w_edge, b_edge, bn_gamma, bn_beta):
    eps = 1e-5
    n, _ = x.shape
    r_e, _ = edge_attr.shape
    dout = w_node.shape[1]

    tiles = max(_cdiv(max(n, r_e), _TILE_ROWS), 1)
    rp = tiles * _TILE_ROWS

    xf = _pad_rows(x.astype(jnp.float32), rp)
    ef = _pad_rows(edge_attr.astype(jnp.float32), rp)
    wn = w_node.astype(jnp.float32)
    we = w_edge.astype(jnp.float32)
    bn = b_node.astype(jnp.float32).reshape(1, dout)
    row = lambda v: v.astype(jnp.float32).reshape(1, dout)
    inv_cnt = 1.0 / float(max(r_e, 1))

    x_enc, s1, s2, e_enc = pl.pallas_call(
        _make_phased_kernel(tiles, inv_cnt, eps),
        out_shape=(jax.ShapeDtypeStruct((rp, dout), jnp.float32),
                   jax.ShapeDtypeStruct((1, dout), jnp.float32),
                   jax.ShapeDtypeStruct((1, dout), jnp.float32),
                   jax.ShapeDtypeStruct((rp, dout), jnp.float32)),
        grid_spec=pltpu.PrefetchScalarGridSpec(
            num_scalar_prefetch=0,
            grid=(2 * tiles,),
            in_specs=[
                pl.BlockSpec((_TILE_ROWS, dout),
                             lambda j: (jnp.minimum(j, tiles - 1), 0)),
                pl.BlockSpec(wn.shape, lambda j: (0, 0)),
                pl.BlockSpec((1, dout), lambda j: (0, 0)),
                pl.BlockSpec((_TILE_ROWS, dout),
                             lambda j: (jnp.where(j < tiles, j, j - tiles), 0)),
                pl.BlockSpec(we.shape, lambda j: (0, 0)),
                pl.BlockSpec((1, dout), lambda j: (0, 0)),
                pl.BlockSpec((1, dout), lambda j: (0, 0)),
            ],
            out_specs=[
                pl.BlockSpec((_TILE_ROWS, dout),
                             lambda j: (jnp.minimum(j, tiles - 1), 0)),
                pl.BlockSpec((1, dout), lambda j: (0, 0)),
                pl.BlockSpec((1, dout), lambda j: (0, 0)),
                pl.BlockSpec((_TILE_ROWS, dout),
                             lambda j: (jnp.maximum(j - tiles, 0), 0)),
            ],
        ),
        compiler_params=pltpu.CompilerParams(
            dimension_semantics=("arbitrary",),
            vmem_limit_bytes=_VMEM_LIMIT_BYTES),
    )(xf, wn, bn, ef, we, row(bn_gamma), row(bn_beta))
    x_enc = x_enc[:n] if rp != n else x_enc
    e_enc = e_enc[:r_e] if rp != r_e else e_enc

    return {"x": x_enc, "edge_attr": e_enc}


# speedup vs baseline: 1.0536x; 1.0536x over previous
import jax
import jax.numpy as jnp
from jax import lax
from jax.experimental import pallas as pl
from jax.experimental.pallas import tpu as pltpu

_TILE_ROWS = 8192
_CACHE_TILES = 10
_VMEM_LIMIT_BYTES = 56 * 1024 * 1024


def _cdiv(a, b):
    return -(-a // b)


def _pad_rows(a, rows_padded):
    r = a.shape[0]
    if rows_padded != r:
        a = jnp.pad(a, ((0, rows_padded - r), (0, 0)))
    return a


def _make_phased_kernel(tiles, cache_tiles, inv_cnt, eps, tile_rows):
    def _phased_kernel(x_ref, wn_ref, bn_ref, e_ref, we_ref, g_ref, b_ref,
                       xo_ref, s1_ref, s2_ref, eo_ref, sc_ref):
        j = pl.program_id(0)

        @pl.when(j == 0)
        def _init():
            s1_ref[...] = jnp.zeros_like(s1_ref)
            s2_ref[...] = jnp.zeros_like(s2_ref)

        def _bn_affine():
            mu = s1_ref[...] * inv_cnt
            var = jnp.maximum(s2_ref[...] * inv_cnt - mu * mu, 0.0)
            scale = g_ref[...] * lax.rsqrt(var + eps)
            shift = b_ref[...] - mu * scale
            return scale, shift

        @pl.when(j < tiles)
        def _phase_a():
            xo_ref[...] = (jnp.dot(x_ref[...], wn_ref[...],
                                   preferred_element_type=jnp.float32)
                           + bn_ref[...]).astype(xo_ref.dtype)
            acc = jnp.dot(e_ref[...], we_ref[...],
                          preferred_element_type=jnp.float32)
            s1_ref[...] += jnp.sum(acc, axis=0, keepdims=True)
            s2_ref[...] += jnp.sum(acc * acc, axis=0, keepdims=True)

            @pl.when(j < cache_tiles)
            def _park():
                sc_ref[pl.ds(j * tile_rows, tile_rows), :] = (
                    acc.astype(sc_ref.dtype))

        @pl.when((j >= tiles) & (j < tiles + cache_tiles))
        def _phase_b_cached():
            k = j - tiles
            scale, shift = _bn_affine()
            enc = sc_ref[pl.ds(k * tile_rows, tile_rows), :].astype(
                jnp.float32)
            eo_ref[...] = (enc * scale + shift).astype(eo_ref.dtype)

        @pl.when(j >= tiles + cache_tiles)
        def _phase_b_fresh():
            scale, shift = _bn_affine()
            acc = jnp.dot(e_ref[...], we_ref[...],
                          preferred_element_type=jnp.float32)
            eo_ref[...] = (acc * scale + shift).astype(eo_ref.dtype)
    return _phased_kernel


def kernel(x, edge_attr, w_node, b_node, w_edge, b_edge, bn_gamma, bn_beta):
    eps = 1e-5
    n, _ = x.shape
    r_e, _ = edge_attr.shape
    dout = w_node.shape[1]

    tiles = max(_cdiv(max(n, r_e), _TILE_ROWS), 1)
    rp = tiles * _TILE_ROWS
    cache_tiles = min(_CACHE_TILES, tiles)

    xf = _pad_rows(x.astype(jnp.float32), rp)
    ef = _pad_rows(edge_attr.astype(jnp.float32), rp)
    wn = w_node.astype(jnp.float32)
    we = w_edge.astype(jnp.float32)
    bn = b_node.astype(jnp.float32).reshape(1, dout)
    row = lambda v: v.astype(jnp.float32).reshape(1, dout)
    inv_cnt = 1.0 / float(max(r_e, 1))

    def _e_index(j):
        in_a = j < tiles
        in_b_cached = j < tiles + cache_tiles
        return (jnp.where(in_a, j,
                          jnp.where(in_b_cached, tiles - 1, j - tiles)), 0)

    x_enc, s1, s2, e_enc = pl.pallas_call(
        _make_phased_kernel(tiles, cache_tiles, inv_cnt, eps, _TILE_ROWS),
        out_shape=(jax.ShapeDtypeStruct((rp, dout), jnp.float32),
                   jax.ShapeDtypeStruct((1, dout), jnp.float32),
                   jax.ShapeDtypeStruct((1, dout), jnp.float32),
                   jax.ShapeDtypeStruct((rp, dout), jnp.float32)),
        grid_spec=pltpu.PrefetchScalarGridSpec(
            num_scalar_prefetch=0,
            grid=(2 * tiles,),
            in_specs=[
                pl.BlockSpec((_TILE_ROWS, dout),
                             lambda j: (jnp.minimum(j, tiles - 1), 0)),
                pl.BlockSpec(wn.shape, lambda j: (0, 0)),
                pl.BlockSpec((1, dout), lambda j: (0, 0)),
                pl.BlockSpec((_TILE_ROWS, dout), _e_index),
                pl.BlockSpec(we.shape, lambda j: (0, 0)),
                pl.BlockSpec((1, dout), lambda j: (0, 0)),
                pl.BlockSpec((1, dout), lambda j: (0, 0)),
            ],
            out_specs=[
                pl.BlockSpec((_TILE_ROWS, dout),
                             lambda j: (jnp.minimum(j, tiles - 1), 0)),
                pl.BlockSpec((1, dout), lambda j: (0, 0)),
                pl.BlockSpec((1, dout), lambda j: (0, 0)),
                pl.BlockSpec((_TILE_ROWS, dout),
                             lambda j: (jnp.maximum(j - tiles, 0), 0)),
            ],
            scratch_shapes=[
                pltpu.VMEM((cache_tiles * _TILE_ROWS, dout), jnp.bfloat16),
            ],
        ),
        compiler_params=pltpu.CompilerParams(
            dimension_semantics=("arbitrary",),
            vmem_limit_bytes=_VMEM_LIMIT_BYTES),
    )(xf, wn, bn, ef, we, row(bn_gamma), row(bn_beta))
    x_enc = x_enc[:n] if rp != n else x_enc
    e_enc = e_enc[:r_e] if rp != r_e else e_enc

    return {"x": x_enc, "edge_attr": e_enc}


# speedup vs baseline: 1.0623x; 1.0082x over previous
import jax
import jax.numpy as jnp
from jax import lax
from jax.experimental import pallas as pl
from jax.experimental.pallas import tpu as pltpu

_TILE_ROWS = 8192
_CACHE_TILES = 11
_VMEM_LIMIT_BYTES = 60000 * 1024


def _cdiv(a, b):
    return -(-a // b)


def _pad_rows(a, rows_padded):
    r = a.shape[0]
    if rows_padded != r:
        a = jnp.pad(a, ((0, rows_padded - r), (0, 0)))
    return a


def _make_phased_kernel(tiles, cache_tiles, inv_cnt, eps, tile_rows):
    def _phased_kernel(x_ref, wn_ref, bn_ref, e_ref, we_ref, g_ref, b_ref,
                       xo_ref, s1_ref, s2_ref, eo_ref, sc_ref):
        j = pl.program_id(0)

        @pl.when(j == 0)
        def _init():
            s1_ref[...] = jnp.zeros_like(s1_ref)
            s2_ref[...] = jnp.zeros_like(s2_ref)

        def _bn_affine():
            mu = s1_ref[...] * inv_cnt
            var = jnp.maximum(s2_ref[...] * inv_cnt - mu * mu, 0.0)
            scale = g_ref[...] * lax.rsqrt(var + eps)
            shift = b_ref[...] - mu * scale
            return scale, shift

        @pl.when(j < tiles)
        def _phase_a():
            xo_ref[...] = (jnp.dot(x_ref[...], wn_ref[...],
                                   preferred_element_type=jnp.float32)
                           + bn_ref[...]).astype(xo_ref.dtype)
            acc = jnp.dot(e_ref[...], we_ref[...],
                          preferred_element_type=jnp.float32)
            s1_ref[...] += jnp.sum(acc, axis=0, keepdims=True)
            s2_ref[...] += jnp.sum(acc * acc, axis=0, keepdims=True)

            @pl.when(j < cache_tiles)
            def _park():
                sc_ref[pl.ds(j * tile_rows, tile_rows), :] = (
                    acc.astype(sc_ref.dtype))

        @pl.when((j >= tiles) & (j < tiles + cache_tiles))
        def _phase_b_cached():
            k = j - tiles
            scale, shift = _bn_affine()
            enc = sc_ref[pl.ds(k * tile_rows, tile_rows), :].astype(
                jnp.float32)
            eo_ref[...] = (enc * scale + shift).astype(eo_ref.dtype)

        @pl.when(j >= tiles + cache_tiles)
        def _phase_b_fresh():
            scale, shift = _bn_affine()
            acc = jnp.dot(e_ref[...], we_ref[...],
                          preferred_element_type=jnp.float32)
            eo_ref[...] = (acc * scale + shift).astype(eo_ref.dtype)
    return _phased_kernel


def kernel(x, edge_attr, w_node, b_node, w_edge, b_edge, bn_gamma, bn_beta):
    eps = 1e-5
    n, _ = x.shape
    r_e, _ = edge_attr.shape
    dout = w_node.shape[1]

    tiles = max(_cdiv(max(n, r_e), _TILE_ROWS), 1)
    rp = tiles * _TILE_ROWS
    cache_tiles = min(_CACHE_TILES, tiles)

    xf = _pad_rows(x.astype(jnp.float32), rp)
    ef = _pad_rows(edge_attr.astype(jnp.float32), rp)
    wn = w_node.astype(jnp.float32)
    we = w_edge.astype(jnp.float32)
    bn = b_node.astype(jnp.float32).reshape(1, dout)
    row = lambda v: v.astype(jnp.float32).reshape(1, dout)
    inv_cnt = 1.0 / float(max(r_e, 1))

    def _e_index(j):
        in_a = j < tiles
        in_b_cached = j < tiles + cache_tiles
        return (jnp.where(in_a, j,
                          jnp.where(in_b_cached, tiles - 1, j - tiles)), 0)

    x_enc, s1, s2, e_enc = pl.pallas_call(
        _make_phased_kernel(tiles, cache_tiles, inv_cnt, eps, _TILE_ROWS),
        out_shape=(jax.ShapeDtypeStruct((rp, dout), jnp.float32),
                   jax.ShapeDtypeStruct((1, dout), jnp.float32),
                   jax.ShapeDtypeStruct((1, dout), jnp.float32),
                   jax.ShapeDtypeStruct((rp, dout), jnp.float32)),
        grid_spec=pltpu.PrefetchScalarGridSpec(
            num_scalar_prefetch=0,
            grid=(2 * tiles,),
            in_specs=[
                pl.BlockSpec((_TILE_ROWS, dout),
                             lambda j: (jnp.minimum(j, tiles - 1), 0)),
                pl.BlockSpec(wn.shape, lambda j: (0, 0)),
                pl.BlockSpec((1, dout), lambda j: (0, 0)),
                pl.BlockSpec((_TILE_ROWS, dout), _e_index),
                pl.BlockSpec(we.shape, lambda j: (0, 0)),
                pl.BlockSpec((1, dout), lambda j: (0, 0)),
                pl.BlockSpec((1, dout), lambda j: (0, 0)),
            ],
            out_specs=[
                pl.BlockSpec((_TILE_ROWS, dout),
                             lambda j: (jnp.minimum(j, tiles - 1), 0)),
                pl.BlockSpec((1, dout), lambda j: (0, 0)),
                pl.BlockSpec((1, dout), lambda j: (0, 0)),
                pl.BlockSpec((_TILE_ROWS, dout),
                             lambda j: (jnp.maximum(j - tiles, 0), 0)),
            ],
            scratch_shapes=[
                pltpu.VMEM((cache_tiles * _TILE_ROWS, dout), jnp.bfloat16),
            ],
        ),
        compiler_params=pltpu.CompilerParams(
            dimension_semantics=("arbitrary",),
            vmem_limit_bytes=_VMEM_LIMIT_BYTES),
    )(xf, wn, bn, ef, we, row(bn_gamma), row(bn_beta))
    x_enc = x_enc[:n] if rp != n else x_enc
    e_enc = e_enc[:r_e] if rp != r_e else e_enc

    return {"x": x_enc, "edge_attr": e_enc}
